# baseline (speedup 1.0000x reference)
import functools

import jax
import jax.numpy as jnp
from jax import lax
from jax.experimental import pallas as pl
from jax.experimental.pallas import tpu as pltpu
from jax.experimental.pallas import tpu_sc as plsc

_N = 10000
_E = 320000
_D = 128
_DH = _D // 2

_NC = 2
_NS = 16
_L = 16
_CH = 64
_CPS = 320
_CPP = 40
_EPAD = _NS * _CH * _CPS
_NBUF = 4
_RPT = (_N // _NS) & ~7
_RTAIL = _N - _NS * _RPT


def _linear_kernel(x_ref, w_ref, b_ref, out_ref):
    acc = lax.dot_general(x_ref[...], w_ref[...],
                          (((1,), (1,)), ((), ())),
                          preferred_element_type=jnp.float32)
    h = jnp.maximum(acc + b_ref[...][None, :], 0.0)
    out_ref[pl.ds(0, _N), :] = h[:, :_DH]
    out_ref[pl.ds(_N, _N), :] = h[:, _DH:]


def _merge_kernel(p_ref, out_ref):
    out_ref[...] = jnp.concatenate(
        [p_ref[pl.ds(0, _N), :], p_ref[pl.ds(_N, _N), :]], axis=1)


def _scale_chunk(rows_v, w_v, k, j):
    @pl.loop(0, _CH // _L)
    def _group(g):
        w16 = w_v[j, pl.ds(g * _L, _L)]
        for e in range(_L):
            wv = w16[e]
            for f in range(_DH // _L):
                sl = pl.ds(f * _L, _L)
                rows_v[k, g * _L + e, sl] = rows_v[k, g * _L + e, sl] * wv


def _sc_edge_kernel(h2_hbm, src_hbm, dst_hbm, w_hbm, z_hbm, out_hbm,
                    src_v, dst_v, w_v, rows_v, h_sh, acc_sh, gsem, ssem):
    c = lax.axis_index("c")
    s = lax.axis_index("s")

    pltpu.sync_copy(h2_hbm.at[pl.ds(c * _N + s * _RPT, _RPT)],
                    h_sh.at[pl.ds(s * _RPT, _RPT)])
    pltpu.sync_copy(z_hbm.at[pl.ds(s * _RPT, _RPT)],
                    acc_sh.at[pl.ds(s * _RPT, _RPT)])

    @pl.when(s == _NS - 1)
    def _tails():
        pltpu.sync_copy(h2_hbm.at[pl.ds(c * _N + _NS * _RPT, _RTAIL)],
                        h_sh.at[pl.ds(_NS * _RPT, _RTAIL)])
        pltpu.sync_copy(z_hbm.at[pl.ds(_NS * _RPT, _RTAIL)],
                        acc_sh.at[pl.ds(_NS * _RPT, _RTAIL)])

    plsc.subcore_barrier()

    def _gather(j, k):
        return pltpu.make_async_copy(
            h_sh.at[src_v.at[j]], rows_v.at[k], gsem.at[k])

    def _scatter(j, k):
        return pltpu.async_copy(
            rows_v.at[k], acc_sh.at[dst_v.at[j]], ssem.at[k], add=True)

    def _scatter_wait(k):
        pltpu.make_async_copy(
            rows_v.at[k], acc_sh.at[dst_v.at[0]], ssem.at[k]).wait()

    for phase in range(_CPS // _CPP):
        row0 = s * _CPS + phase * _CPP
        pltpu.sync_copy(src_hbm.at[pl.ds(row0, _CPP)], src_v)
        pltpu.sync_copy(dst_hbm.at[pl.ds(row0, _CPP)], dst_v)
        pltpu.sync_copy(w_hbm.at[pl.ds(row0, _CPP)], w_v)

        _gather(0, 0).start()
        _gather(1, 1).start()

        @pl.loop(0, _CPP, step=_NBUF)
        def _chunk(i):
            for k in range(_NBUF):
                j = i + k

                @pl.when(j >= 2)
                def _drain():
                    _scatter_wait((k + 2) % _NBUF)

                @pl.when(j + 2 < _CPP)
                def _prefetch():
                    _gather(j + 2, (k + 2) % _NBUF).start()

                _gather(j, k).wait()
                _scale_chunk(rows_v, w_v, k, j)
                _scatter(j, k)

        _scatter_wait((_CPP - 2) % _NBUF)
        _scatter_wait((_CPP - 1) % _NBUF)

    plsc.subcore_barrier()
    pltpu.sync_copy(acc_sh.at[pl.ds(s * _RPT, _RPT)],
                    out_hbm.at[pl.ds(c * _N + s * _RPT, _RPT)])

    @pl.when(s == _NS - 1)
    def _out_tail():
        pltpu.sync_copy(acc_sh.at[pl.ds(_NS * _RPT, _RTAIL)],
                        out_hbm.at[pl.ds(c * _N + _NS * _RPT, _RTAIL)])


@functools.lru_cache(maxsize=None)
def _sc_edge():
    return pl.kernel(
        _sc_edge_kernel,
        out_type=jax.ShapeDtypeStruct((_NC * _N, _DH), jnp.float32),
        mesh=plsc.VectorSubcoreMesh(core_axis_name="c", subcore_axis_name="s",
                                    num_cores=_NC, num_subcores=_NS),
        compiler_params=pltpu.CompilerParams(use_tc_tiling_on_sc=False),
        scratch_types=[
            pltpu.VMEM((_CPP, _CH), jnp.int32),
            pltpu.VMEM((_CPP, _CH), jnp.int32),
            pltpu.VMEM((_CPP, _CH), jnp.float32),
            pltpu.VMEM((_NBUF, _CH, _DH), jnp.float32),
            pltpu.VMEM_SHARED((_N, _DH), jnp.float32),
            pltpu.VMEM_SHARED((_N, _DH), jnp.float32),
            pltpu.SemaphoreType.DMA((_NBUF,)),
            pltpu.SemaphoreType.DMA((_NBUF,)),
        ],
    )


def kernel(x, edge_index, edge_w, W, b):
    h2 = pl.pallas_call(
        _linear_kernel,
        out_shape=jax.ShapeDtypeStruct((_NC * _N, _DH), jnp.float32),
    )(x, W, b)

    pad = _EPAD - _E
    fill = (jnp.arange(pad, dtype=jnp.int32) * 31) % _N
    src = jnp.concatenate([edge_index[0], fill]).reshape(_NS * _CPS, _CH)
    dst = jnp.concatenate([edge_index[1], fill]).reshape(_NS * _CPS, _CH)
    w = jnp.pad(edge_w, (0, pad)).reshape(_NS * _CPS, _CH)

    zeros = jnp.zeros((_N, _DH), jnp.float32)
    o2 = _sc_edge()(h2, src, dst, w, zeros)

    out = pl.pallas_call(
        _merge_kernel,
        out_shape=jax.ShapeDtypeStruct((_N, _D), jnp.float32),
    )(o2)
    return out


# speedup vs baseline: 1.3353x; 1.3353x over previous
import functools

import jax
import jax.numpy as jnp
from jax import lax
from jax.experimental import pallas as pl
from jax.experimental.pallas import tpu as pltpu
from jax.experimental.pallas import tpu_sc as plsc

_N = 10000
_E = 320000
_D = 128
_DH = _D // 2

_NC = 2
_NS = 16
_L = 16
_CH = 128
_CPS = 160
_CPP = 40
_EPAD = _NS * _CH * _CPS
_NBUF = 4
_RPT = (_N // _NS) & ~7
_RTAIL = _N - _NS * _RPT


def _linear_kernel(x_ref, w_ref, b_ref, out_ref):
    acc = lax.dot_general(x_ref[...], w_ref[...],
                          (((1,), (1,)), ((), ())),
                          preferred_element_type=jnp.float32)
    h = jnp.maximum(acc + b_ref[...][None, :], 0.0)
    out_ref[pl.ds(0, _N), :] = h[:, :_DH]
    out_ref[pl.ds(_N, _N), :] = h[:, _DH:]


def _merge_kernel(p_ref, out_ref):
    out_ref[...] = jnp.concatenate(
        [p_ref[pl.ds(0, _N), :], p_ref[pl.ds(_N, _N), :]], axis=1)


def _scale_chunk(rows_v, w_v, k, j):
    @pl.loop(0, _CH // _L)
    def _group(g):
        w16 = w_v[j, pl.ds(g * _L, _L)]
        for e in range(_L):
            wv = w16[e]
            for f in range(_DH // _L):
                sl = pl.ds(f * _L, _L)
                rows_v[k, g * _L + e, sl] = rows_v[k, g * _L + e, sl] * wv


def _sc_edge_kernel(h2_hbm, src_hbm, dst_hbm, w_hbm, z_hbm, out_hbm,
                    src_v, dst_v, w_v, rows_v, h_sh, acc_sh, gsem, ssem):
    c = lax.axis_index("c")
    s = lax.axis_index("s")

    pltpu.sync_copy(h2_hbm.at[pl.ds(c * _N + s * _RPT, _RPT)],
                    h_sh.at[pl.ds(s * _RPT, _RPT)])
    pltpu.sync_copy(z_hbm.at[pl.ds(s * _RPT, _RPT)],
                    acc_sh.at[pl.ds(s * _RPT, _RPT)])

    @pl.when(s == _NS - 1)
    def _tails():
        pltpu.sync_copy(h2_hbm.at[pl.ds(c * _N + _NS * _RPT, _RTAIL)],
                        h_sh.at[pl.ds(_NS * _RPT, _RTAIL)])
        pltpu.sync_copy(z_hbm.at[pl.ds(_NS * _RPT, _RTAIL)],
                        acc_sh.at[pl.ds(_NS * _RPT, _RTAIL)])

    plsc.subcore_barrier()

    def _gather(j, k):
        return pltpu.make_async_copy(
            h_sh.at[src_v.at[j]], rows_v.at[k], gsem.at[k])

    def _scatter(j, k):
        return pltpu.async_copy(
            rows_v.at[k], acc_sh.at[dst_v.at[j]], ssem.at[k], add=True)

    def _scatter_wait(k):
        pltpu.make_async_copy(
            rows_v.at[k], acc_sh.at[dst_v.at[0]], ssem.at[k]).wait()

    for phase in range(_CPS // _CPP):
        row0 = s * _CPS + phase * _CPP
        pltpu.sync_copy(src_hbm.at[pl.ds(row0, _CPP)], src_v)
        pltpu.sync_copy(dst_hbm.at[pl.ds(row0, _CPP)], dst_v)
        pltpu.sync_copy(w_hbm.at[pl.ds(row0, _CPP)], w_v)

        _gather(0, 0).start()
        _gather(1, 1).start()

        @pl.loop(0, _CPP, step=_NBUF)
        def _chunk(i):
            for k in range(_NBUF):
                j = i + k

                @pl.when(j >= 2)
                def _drain():
                    _scatter_wait((k + 2) % _NBUF)

                @pl.when(j + 2 < _CPP)
                def _prefetch():
                    _gather(j + 2, (k + 2) % _NBUF).start()

                _gather(j, k).wait()
                _scale_chunk(rows_v, w_v, k, j)
                _scatter(j, k)

        _scatter_wait((_CPP - 2) % _NBUF)
        _scatter_wait((_CPP - 1) % _NBUF)

    plsc.subcore_barrier()
    pltpu.sync_copy(acc_sh.at[pl.ds(s * _RPT, _RPT)],
                    out_hbm.at[pl.ds(c * _N + s * _RPT, _RPT)])

    @pl.when(s == _NS - 1)
    def _out_tail():
        pltpu.sync_copy(acc_sh.at[pl.ds(_NS * _RPT, _RTAIL)],
                        out_hbm.at[pl.ds(c * _N + _NS * _RPT, _RTAIL)])


@functools.lru_cache(maxsize=None)
def _sc_edge():
    return pl.kernel(
        _sc_edge_kernel,
        out_type=jax.ShapeDtypeStruct((_NC * _N, _DH), jnp.float32),
        mesh=plsc.VectorSubcoreMesh(core_axis_name="c", subcore_axis_name="s",
                                    num_cores=_NC, num_subcores=_NS),
        compiler_params=pltpu.CompilerParams(use_tc_tiling_on_sc=False),
        scratch_types=[
            pltpu.VMEM((_CPP, _CH), jnp.int32),
            pltpu.VMEM((_CPP, _CH), jnp.int32),
            pltpu.VMEM((_CPP, _CH), jnp.float32),
            pltpu.VMEM((_NBUF, _CH, _DH), jnp.float32),
            pltpu.VMEM_SHARED((_N, _DH), jnp.float32),
            pltpu.VMEM_SHARED((_N, _DH), jnp.float32),
            pltpu.SemaphoreType.DMA((_NBUF,)),
            pltpu.SemaphoreType.DMA((_NBUF,)),
        ],
    )


def kernel(x, edge_index, edge_w, W, b):
    h2 = pl.pallas_call(
        _linear_kernel,
        out_shape=jax.ShapeDtypeStruct((_NC * _N, _DH), jnp.float32),
    )(x, W, b)

    pad = _EPAD - _E
    fill = (jnp.arange(pad, dtype=jnp.int32) * 31) % _N
    src = jnp.concatenate([edge_index[0], fill]).reshape(_NS * _CPS, _CH)
    dst = jnp.concatenate([edge_index[1], fill]).reshape(_NS * _CPS, _CH)
    w = jnp.pad(edge_w, (0, pad)).reshape(_NS * _CPS, _CH)

    zeros = jnp.zeros((_N, _DH), jnp.float32)
    o2 = _sc_edge()(h2, src, dst, w, zeros)

    out = pl.pallas_call(
        _merge_kernel,
        out_shape=jax.ShapeDtypeStruct((_N, _D), jnp.float32),
    )(o2)
    return out


# speedup vs baseline: 1.4524x; 1.0877x over previous
import functools

import jax
import jax.numpy as jnp
from jax import lax
from jax.experimental import pallas as pl
from jax.experimental.pallas import tpu as pltpu
from jax.experimental.pallas import tpu_sc as plsc

_N = 10000
_E = 320000
_D = 128
_DH = _D // 2

_NC = 2
_NS = 16
_L = 16
_CH = 128
_CPS = 160
_CPP = 40
_EPAD = _NS * _CH * _CPS
_NBUF = 4
_RPT = (_N // _NS) & ~7
_RTAIL = _N - _NS * _RPT


def _linear_kernel(x_ref, w_ref, b_ref, out_ref):
    acc = lax.dot_general(x_ref[...], w_ref[...],
                          (((1,), (1,)), ((), ())),
                          preferred_element_type=jnp.float32)
    out_ref[...] = jnp.maximum(acc + b_ref[...][None, :], 0.0)


def _scale_chunk(rows_v, w_v, k, j):
    @pl.loop(0, _CH // _L)
    def _group(g):
        w16 = w_v[j, pl.ds(g * _L, _L)]
        for e in range(_L):
            wv = w16[e]
            for f in range(_DH // _L):
                sl = pl.ds(f * _L, _L)
                rows_v[k, g * _L + e, sl] = rows_v[k, g * _L + e, sl] * wv


def _sc_edge_kernel(h_hbm, src_hbm, dst_hbm, w_hbm, z_hbm, out_hbm,
                    src_v, dst_v, w_v, rows_v, h_sh, acc_sh, gsem, ssem):
    c = lax.axis_index("c")
    s = lax.axis_index("s")

    for cc in range(_NC):
        @pl.when(c == cc)
        def _stage():
            pltpu.sync_copy(
                h_hbm.at[pl.ds(s * _RPT, _RPT), pl.ds(cc * _DH, _DH)],
                h_sh.at[pl.ds(s * _RPT, _RPT)])

            @pl.when(s == _NS - 1)
            def _t():
                pltpu.sync_copy(
                    h_hbm.at[pl.ds(_NS * _RPT, _RTAIL), pl.ds(cc * _DH, _DH)],
                    h_sh.at[pl.ds(_NS * _RPT, _RTAIL)])

    pltpu.sync_copy(z_hbm.at[pl.ds(s * _RPT, _RPT)],
                    acc_sh.at[pl.ds(s * _RPT, _RPT)])

    @pl.when(s == _NS - 1)
    def _tails():
        pltpu.sync_copy(z_hbm.at[pl.ds(_NS * _RPT, _RTAIL)],
                        acc_sh.at[pl.ds(_NS * _RPT, _RTAIL)])

    plsc.subcore_barrier()

    def _gather(j, k):
        return pltpu.make_async_copy(
            h_sh.at[src_v.at[j]], rows_v.at[k], gsem.at[k])

    def _scatter(j, k):
        return pltpu.async_copy(
            rows_v.at[k], acc_sh.at[dst_v.at[j]], ssem.at[k], add=True)

    def _scatter_wait(k):
        pltpu.make_async_copy(
            rows_v.at[k], acc_sh.at[dst_v.at[0]], ssem.at[k]).wait()

    for phase in range(_CPS // _CPP):
        row0 = s * _CPS + phase * _CPP
        pltpu.sync_copy(src_hbm.at[pl.ds(row0, _CPP)], src_v)
        pltpu.sync_copy(dst_hbm.at[pl.ds(row0, _CPP)], dst_v)
        pltpu.sync_copy(w_hbm.at[pl.ds(row0, _CPP)], w_v)

        _gather(0, 0).start()
        _gather(1, 1).start()

        @pl.loop(0, _CPP, step=_NBUF)
        def _chunk(i):
            for k in range(_NBUF):
                j = i + k

                @pl.when(j >= 2)
                def _drain():
                    _scatter_wait((k + 2) % _NBUF)

                @pl.when(j + 2 < _CPP)
                def _prefetch():
                    _gather(j + 2, (k + 2) % _NBUF).start()

                _gather(j, k).wait()
                _scale_chunk(rows_v, w_v, k, j)
                _scatter(j, k)

        _scatter_wait((_CPP - 2) % _NBUF)
        _scatter_wait((_CPP - 1) % _NBUF)

    plsc.subcore_barrier()
    for cc in range(_NC):
        @pl.when(c == cc)
        def _copy_out():
            pltpu.sync_copy(
                acc_sh.at[pl.ds(s * _RPT, _RPT)],
                out_hbm.at[pl.ds(s * _RPT, _RPT), pl.ds(cc * _DH, _DH)])

            @pl.when(s == _NS - 1)
            def _t():
                pltpu.sync_copy(
                    acc_sh.at[pl.ds(_NS * _RPT, _RTAIL)],
                    out_hbm.at[pl.ds(_NS * _RPT, _RTAIL), pl.ds(cc * _DH, _DH)])


@functools.lru_cache(maxsize=None)
def _sc_edge():
    return pl.kernel(
        _sc_edge_kernel,
        out_type=jax.ShapeDtypeStruct((_N, _D), jnp.float32),
        mesh=plsc.VectorSubcoreMesh(core_axis_name="c", subcore_axis_name="s",
                                    num_cores=_NC, num_subcores=_NS),
        compiler_params=pltpu.CompilerParams(use_tc_tiling_on_sc=False),
        scratch_types=[
            pltpu.VMEM((_CPP, _CH), jnp.int32),
            pltpu.VMEM((_CPP, _CH), jnp.int32),
            pltpu.VMEM((_CPP, _CH), jnp.float32),
            pltpu.VMEM((_NBUF, _CH, _DH), jnp.float32),
            pltpu.VMEM_SHARED((_N, _DH), jnp.float32),
            pltpu.VMEM_SHARED((_N, _DH), jnp.float32),
            pltpu.SemaphoreType.DMA((_NBUF,)),
            pltpu.SemaphoreType.DMA((_NBUF,)),
        ],
    )


def kernel(x, edge_index, edge_w, W, b):
    h = pl.pallas_call(
        _linear_kernel,
        out_shape=jax.ShapeDtypeStruct((_N, _D), jnp.float32),
    )(x, W, b)

    pad = _EPAD - _E
    fill = (jnp.arange(pad, dtype=jnp.int32) * 31) % _N
    src = jnp.concatenate([edge_index[0], fill]).reshape(_NS * _CPS, _CH)
    dst = jnp.concatenate([edge_index[1], fill]).reshape(_NS * _CPS, _CH)
    w = jnp.pad(edge_w, (0, pad)).reshape(_NS * _CPS, _CH)

    zeros = jnp.zeros((_N, _DH), jnp.float32)
    return _sc_edge()(h, src, dst, w, zeros)
